# Initial kernel scaffold; baseline (speedup 1.0000x reference)
import jax
import jax.numpy as jnp
from jax.experimental import pallas as pl


def kernel(x, edge_index, W1, b1, Wc1, bc1, Wc2, bc2, Wc3, bc3, W4, b4):
    raise NotImplementedError("write your pallas kernel here")


# speedup vs baseline: 44.7307x; 44.7307x over previous
import jax
import jax.numpy as jnp
from jax import lax
from jax.experimental import pallas as pl
from jax.experimental.pallas import tpu as pltpu
from jax.experimental.pallas import tpu_sc as plsc

N, D, E, H, O = 10000, 128, 320000, 32, 128

_NCORES, _NSUB = 2, 16
NW = _NCORES * _NSUB
K = 128
GK = 10
CH_A = 80
CH_B = 80
NG_A = CH_A // GK
NG_B = CH_B // GK
EP_A = _NSUB * CH_A * K
EP_B = _NSUB * CH_B * K
EP = EP_A + EP_B
RPT = 632
R = RPT * _NSUB
BN = 2000
_HIGH = lax.Precision.HIGHEST

_sc_mesh = plsc.VectorSubcoreMesh(
    core_axis_name="c", subcore_axis_name="s",
    num_cores=_NCORES, num_subcores=_NSUB)
_sc_params = pltpu.CompilerParams(use_tc_tiling_on_sc=False)


def _load_core_indices(c, s, a_hbm, b_hbm, vref):
    @pl.when(c == 0)
    def _():
        pltpu.sync_copy(a_hbm.at[s], vref.at[pl.ds(0, CH_A)])

    @pl.when(c == 1)
    def _():
        pltpu.sync_copy(b_hbm.at[s], vref.at[pl.ds(0, CH_B)])


def _deg_body(dstA, dstB, ones_hbm, zero_hbm, out_hbm, dst_v, ones_v, acc):
    c = lax.axis_index("c")
    s = lax.axis_index("s")
    pltpu.sync_copy(zero_hbm.at[pl.ds(s * RPT, RPT), :],
                    acc.at[pl.ds(s * RPT, RPT), :])
    pltpu.sync_copy(ones_hbm, ones_v)
    _load_core_indices(c, s, dstA, dstB, dst_v)
    plsc.subcore_barrier()

    def body(i, carry):
        pltpu.sync_copy(ones_v, acc.at[dst_v.at[i]], add=True)
        return carry

    lax.fori_loop(0, jnp.where(c == 0, CH_A, CH_B), body, 0)
    plsc.subcore_barrier()
    pltpu.sync_copy(acc.at[pl.ds(s * RPT, RPT), :],
                    out_hbm.at[c, pl.ds(s * RPT, RPT), :])


_deg = pl.kernel(
    _deg_body,
    out_type=jax.ShapeDtypeStruct((2, R, 8), jnp.float32),
    mesh=_sc_mesh,
    scratch_types=[
        pltpu.VMEM((max(CH_A, CH_B), K), jnp.int32),
        pltpu.VMEM((K, 8), jnp.float32),
        pltpu.VMEM_SHARED((R, 8), jnp.float32),
    ],
    name="sc_degree",
    compiler_params=_sc_params,
)


def _prop_body(y_hbm, srcA, dstA, srcB, dstB, zero_hbm, out_hbm,
               src_v, dst_v, rows_v, acc, gsem0, gsem1, ssem):
    c = lax.axis_index("c")
    s = lax.axis_index("s")
    pltpu.sync_copy(zero_hbm.at[pl.ds(s * RPT, RPT), :],
                    acc.at[pl.ds(s * RPT, RPT), :])
    _load_core_indices(c, s, srcA, srcB, src_v)
    _load_core_indices(c, s, dstA, dstB, dst_v)
    plsc.subcore_barrier()

    ng = jnp.where(c == 0, NG_A, NG_B)

    def fire_g(g, bank, sem):
        return [pltpu.async_copy(y_hbm.at[src_v.at[g * GK + b]],
                                 rows_v.at[bank, b], sem)
                for b in range(GK)]

    def drain_g(bank, sem):
        for b in range(GK):
            pltpu.make_async_copy(y_hbm.at[src_v.at[b]],
                                  rows_v.at[bank, b], sem).wait()

    def fire_s(g, bank, sem):
        return [pltpu.async_copy(rows_v.at[bank, b],
                                 acc.at[dst_v.at[g * GK + b]], sem, add=True)
                for b in range(GK)]

    fire_g(0, 0, gsem0)

    def body(j, carry):
        g0 = 2 * j
        g1 = g0 + 1
        fire_g(g1, 1, gsem1)
        drain_g(0, gsem0)
        for cp in fire_s(g0, 0, ssem):
            cp.wait()
        fire_g(jnp.minimum(g0 + 2, ng - 1), 0, gsem0)
        drain_g(1, gsem1)
        for cp in fire_s(g1, 1, ssem):
            cp.wait()
        return carry

    lax.fori_loop(0, ng // 2, body, 0)
    drain_g(0, gsem0)
    plsc.subcore_barrier()
    pltpu.sync_copy(acc.at[pl.ds(s * RPT, RPT), :],
                    out_hbm.at[c, pl.ds(s * RPT, RPT), :])


_propagate = pl.kernel(
    _prop_body,
    out_type=jax.ShapeDtypeStruct((2, R, H), jnp.float32),
    mesh=_sc_mesh,
    scratch_types=[
        pltpu.VMEM((max(CH_A, CH_B), K), jnp.int32),
        pltpu.VMEM((max(CH_A, CH_B), K), jnp.int32),
        pltpu.VMEM((2, GK, K, H), jnp.float32),
        pltpu.VMEM_SHARED((R, H), jnp.float32),
        pltpu.SemaphoreType.DMA,
        pltpu.SemaphoreType.DMA,
        pltpu.SemaphoreType.DMA,
    ],
    name="sc_propagate",
    compiler_params=_sc_params,
)


def _mynorm(t):
    mn = jnp.min(t, axis=1, keepdims=True)
    mx = jnp.max(t, axis=1, keepdims=True)
    return 2.0 * (t - mn) / (mx - mn + 1e-08) - 1.0


def _dot_t(a, b):
    return lax.dot_general(a, b, (((1,), (1,)), ((), ())),
                           preferred_element_type=jnp.float32)


def _t1_body(x_ref, w1_ref, b1_ref, degp_ref, x0_ref, y0_ref, dinv_ref):
    t = jnp.maximum(_dot_t(x_ref[...], w1_ref[...]) + b1_ref[...], 0.0)
    x0 = _mynorm(t)
    deg = degp_ref[0, :, 0:1] + degp_ref[1, :, 0:1] + 1.0
    dinv = lax.rsqrt(deg)
    x0_ref[...] = x0
    y0_ref[...] = x0 * dinv
    dinv_ref[...] = jnp.broadcast_to(dinv, (BN, 8))


def _t2_body(sp_ref, yprev_ref, dinv_ref, wc_ref, bc_ref, xout_ref, ynext_ref):
    dinv = dinv_ref[:, 0:1]
    agg = (sp_ref[0] + sp_ref[1] + yprev_ref[...]) * dinv
    xo = _dot_t(agg, wc_ref[...]) + bc_ref[...]
    xout_ref[...] = xo
    ynext_ref[...] = xo * dinv


def _t3_body(sp_ref, yprev_ref, dinv_ref, wc_ref, bc_ref, x0_ref,
             xx_ref, ynext_ref):
    dinv = dinv_ref[:, 0:1]
    agg = (sp_ref[0] + sp_ref[1] + yprev_ref[...]) * dinv
    x2 = _dot_t(agg, wc_ref[...]) + bc_ref[...]
    xx_ref[...] = _mynorm(x2) - _mynorm(x0_ref[...])
    ynext_ref[...] = x2 * dinv


def _t4_body(sp_ref, yprev_ref, dinv_ref, wc_ref, bc_ref, x0_ref, x1_ref,
             xx2_ref, w4_ref, b4_ref, out_ref):
    dinv = dinv_ref[:, 0:1]
    agg = (sp_ref[0] + sp_ref[1] + yprev_ref[...]) * dinv
    x3 = _dot_t(agg, wc_ref[...]) + bc_ref[...]
    xx3 = _mynorm(x3) - _mynorm(x1_ref[...])
    cat = jnp.concatenate([x0_ref[...], x1_ref[...], xx2_ref[...], xx3],
                          axis=1)
    out_ref[...] = _dot_t(cat, w4_ref[...]) + b4_ref[...]


def _node_spec(width):
    return pl.BlockSpec((BN, width), lambda i: (i, 0))


def _full_spec(shape):
    nd = len(shape)
    return pl.BlockSpec(shape, lambda i, _nd=nd: (0,) * _nd)


def _part_spec(width):
    return pl.BlockSpec((2, BN, width), lambda i: (0, i, 0))


_GRID = N // BN


def _nodes_out(widths):
    return [jax.ShapeDtypeStruct((N, w), jnp.float32) for w in widths]


_t1 = pl.pallas_call(
    _t1_body,
    grid=(_GRID,),
    in_specs=[_node_spec(D), _full_spec((H, D)), _full_spec((1, H)),
              _part_spec(8)],
    out_specs=[_node_spec(H), _node_spec(H), _node_spec(8)],
    out_shape=_nodes_out([H, H, 8]),
)

_t2 = pl.pallas_call(
    _t2_body,
    grid=(_GRID,),
    in_specs=[_part_spec(H), _node_spec(H), _node_spec(8),
              _full_spec((H, H)), _full_spec((1, H))],
    out_specs=[_node_spec(H), _node_spec(H)],
    out_shape=_nodes_out([H, H]),
)

_t3 = pl.pallas_call(
    _t3_body,
    grid=(_GRID,),
    in_specs=[_part_spec(H), _node_spec(H), _node_spec(8),
              _full_spec((H, H)), _full_spec((1, H)), _node_spec(H)],
    out_specs=[_node_spec(H), _node_spec(H)],
    out_shape=_nodes_out([H, H]),
)

_t4 = pl.pallas_call(
    _t4_body,
    grid=(_GRID,),
    in_specs=[_part_spec(H), _node_spec(H), _node_spec(8),
              _full_spec((H, H)), _full_spec((1, H)), _node_spec(H),
              _node_spec(H), _node_spec(H), _full_spec((O, 4 * H)),
              _full_spec((1, O))],
    out_specs=_node_spec(O),
    out_shape=jax.ShapeDtypeStruct((N, O), jnp.float32),
)


def kernel(x, edge_index, W1, b1, Wc1, bc1, Wc2, bc2, Wc3, bc3, W4, b4):
    pad = EP - E
    pad_i = jnp.arange(pad, dtype=edge_index.dtype)
    srcp = jnp.concatenate([edge_index[0], pad_i % N])
    dstp = jnp.concatenate([edge_index[1], N + pad_i % (R - N)])
    srcA = srcp[:EP_A].reshape(_NSUB, CH_A, K)
    srcB = srcp[EP_A:].reshape(_NSUB, CH_B, K)
    dstA = dstp[:EP_A].reshape(_NSUB, CH_A, K)
    dstB = dstp[EP_A:].reshape(_NSUB, CH_B, K)
    z32 = jnp.zeros((R, H), jnp.float32)
    z8 = jnp.zeros((R, 8), jnp.float32)
    ones8 = jnp.ones((K, 8), jnp.float32)

    degp = _deg(dstA, dstB, ones8, z8)
    x0, y0, dinv = _t1(x, W1, b1.reshape(1, H), degp)
    sp1 = _propagate(y0, srcA, dstA, srcB, dstB, z32)
    x1, y1 = _t2(sp1, y0, dinv, Wc1, bc1.reshape(1, H))
    sp2 = _propagate(y1, srcA, dstA, srcB, dstB, z32)
    xx2, y2 = _t3(sp2, y1, dinv, Wc2, bc2.reshape(1, H), x0)
    sp3 = _propagate(y2, srcA, dstA, srcB, dstB, z32)
    x4 = _t4(sp3, y2, dinv, Wc3, bc3.reshape(1, H), x0, x1, xx2,
             W4, b4.reshape(1, O))
    return x4


# speedup vs baseline: 45.1194x; 1.0087x over previous
import jax
import jax.numpy as jnp
from jax import lax
from jax.experimental import pallas as pl
from jax.experimental.pallas import tpu as pltpu
from jax.experimental.pallas import tpu_sc as plsc

N, D, E, H, O = 10000, 128, 320000, 32, 128

_NCORES, _NSUB = 2, 16
NW = _NCORES * _NSUB
K = 128
GK = 8
CH_A = 80
CH_B = 80
NG_A = CH_A // GK
NG_B = CH_B // GK
EP_A = _NSUB * CH_A * K
EP_B = _NSUB * CH_B * K
EP = EP_A + EP_B
RPT = 632
R = RPT * _NSUB
BN = 2000
_HIGH = lax.Precision.HIGHEST

_sc_mesh = plsc.VectorSubcoreMesh(
    core_axis_name="c", subcore_axis_name="s",
    num_cores=_NCORES, num_subcores=_NSUB)
_sc_params = pltpu.CompilerParams(use_tc_tiling_on_sc=False)


def _load_core_indices(c, s, a_hbm, b_hbm, vref):
    @pl.when(c == 0)
    def _():
        pltpu.sync_copy(a_hbm.at[s], vref.at[pl.ds(0, CH_A)])

    @pl.when(c == 1)
    def _():
        pltpu.sync_copy(b_hbm.at[s], vref.at[pl.ds(0, CH_B)])


def _deg_body(dstA, dstB, ones_hbm, zero_hbm, out_hbm, dst_v, ones_v, acc):
    c = lax.axis_index("c")
    s = lax.axis_index("s")
    pltpu.sync_copy(zero_hbm.at[pl.ds(s * RPT, RPT), :],
                    acc.at[pl.ds(s * RPT, RPT), :])
    pltpu.sync_copy(ones_hbm, ones_v)
    _load_core_indices(c, s, dstA, dstB, dst_v)
    plsc.subcore_barrier()

    def body(i, carry):
        pltpu.sync_copy(ones_v, acc.at[dst_v.at[i]], add=True)
        return carry

    lax.fori_loop(0, jnp.where(c == 0, CH_A, CH_B), body, 0)
    plsc.subcore_barrier()
    pltpu.sync_copy(acc.at[pl.ds(s * RPT, RPT), :],
                    out_hbm.at[c, pl.ds(s * RPT, RPT), :])


_deg = pl.kernel(
    _deg_body,
    out_type=jax.ShapeDtypeStruct((2, R, 8), jnp.float32),
    mesh=_sc_mesh,
    scratch_types=[
        pltpu.VMEM((max(CH_A, CH_B), K), jnp.int32),
        pltpu.VMEM((K, 8), jnp.float32),
        pltpu.VMEM_SHARED((R, 8), jnp.float32),
    ],
    name="sc_degree",
    compiler_params=_sc_params,
)


def _prop_body(y_hbm, srcA, dstA, srcB, dstB, zero_hbm, out_hbm,
               src_v, dst_v, rows_v, acc, gsem0, gsem1, ssem):
    c = lax.axis_index("c")
    s = lax.axis_index("s")
    pltpu.sync_copy(zero_hbm.at[pl.ds(s * RPT, RPT), :],
                    acc.at[pl.ds(s * RPT, RPT), :])
    _load_core_indices(c, s, srcA, srcB, src_v)
    _load_core_indices(c, s, dstA, dstB, dst_v)
    plsc.subcore_barrier()

    ng = jnp.where(c == 0, NG_A, NG_B)

    def fire_g(g, bank, sem):
        return [pltpu.async_copy(y_hbm.at[src_v.at[g * GK + b]],
                                 rows_v.at[bank, b], sem)
                for b in range(GK)]

    def drain_g(bank, sem):
        for b in range(GK):
            pltpu.make_async_copy(y_hbm.at[src_v.at[b]],
                                  rows_v.at[bank, b], sem).wait()

    def fire_s(g, bank, sem):
        return [pltpu.async_copy(rows_v.at[bank, b],
                                 acc.at[dst_v.at[g * GK + b]], sem, add=True)
                for b in range(GK)]

    fire_g(0, 0, gsem0)

    def body(j, carry):
        g0 = 2 * j
        g1 = g0 + 1
        fire_g(g1, 1, gsem1)
        drain_g(0, gsem0)
        for cp in fire_s(g0, 0, ssem):
            cp.wait()
        fire_g(jnp.minimum(g0 + 2, ng - 1), 0, gsem0)
        drain_g(1, gsem1)
        for cp in fire_s(g1, 1, ssem):
            cp.wait()
        return carry

    lax.fori_loop(0, ng // 2, body, 0)
    drain_g(0, gsem0)
    plsc.subcore_barrier()
    pltpu.sync_copy(acc.at[pl.ds(s * RPT, RPT), :],
                    out_hbm.at[c, pl.ds(s * RPT, RPT), :])


_propagate = pl.kernel(
    _prop_body,
    out_type=jax.ShapeDtypeStruct((2, R, H), jnp.float32),
    mesh=_sc_mesh,
    scratch_types=[
        pltpu.VMEM((max(CH_A, CH_B), K), jnp.int32),
        pltpu.VMEM((max(CH_A, CH_B), K), jnp.int32),
        pltpu.VMEM((2, GK, K, H), jnp.float32),
        pltpu.VMEM_SHARED((R, H), jnp.float32),
        pltpu.SemaphoreType.DMA,
        pltpu.SemaphoreType.DMA,
        pltpu.SemaphoreType.DMA,
    ],
    name="sc_propagate",
    compiler_params=_sc_params,
)


def _mynorm(t):
    mn = jnp.min(t, axis=1, keepdims=True)
    mx = jnp.max(t, axis=1, keepdims=True)
    return 2.0 * (t - mn) / (mx - mn + 1e-08) - 1.0


def _dot_t(a, b):
    return lax.dot_general(a, b, (((1,), (1,)), ((), ())),
                           preferred_element_type=jnp.float32)


def _t1_body(x_ref, w1_ref, b1_ref, degp_ref, x0_ref, y0_ref, dinv_ref):
    t = jnp.maximum(_dot_t(x_ref[...], w1_ref[...]) + b1_ref[...], 0.0)
    x0 = _mynorm(t)
    deg = degp_ref[0, :, 0:1] + degp_ref[1, :, 0:1] + 1.0
    dinv = lax.rsqrt(deg)
    x0_ref[...] = x0
    y0_ref[...] = x0 * dinv
    dinv_ref[...] = jnp.broadcast_to(dinv, (BN, 8))


def _t2_body(sp_ref, yprev_ref, dinv_ref, wc_ref, bc_ref, xout_ref, ynext_ref):
    dinv = dinv_ref[:, 0:1]
    agg = (sp_ref[0] + sp_ref[1] + yprev_ref[...]) * dinv
    xo = _dot_t(agg, wc_ref[...]) + bc_ref[...]
    xout_ref[...] = xo
    ynext_ref[...] = xo * dinv


def _t3_body(sp_ref, yprev_ref, dinv_ref, wc_ref, bc_ref, x0_ref,
             xx_ref, ynext_ref):
    dinv = dinv_ref[:, 0:1]
    agg = (sp_ref[0] + sp_ref[1] + yprev_ref[...]) * dinv
    x2 = _dot_t(agg, wc_ref[...]) + bc_ref[...]
    xx_ref[...] = _mynorm(x2) - _mynorm(x0_ref[...])
    ynext_ref[...] = x2 * dinv


def _t4_body(sp_ref, yprev_ref, dinv_ref, wc_ref, bc_ref, x0_ref, x1_ref,
             xx2_ref, w4_ref, b4_ref, out_ref):
    dinv = dinv_ref[:, 0:1]
    agg = (sp_ref[0] + sp_ref[1] + yprev_ref[...]) * dinv
    x3 = _dot_t(agg, wc_ref[...]) + bc_ref[...]
    xx3 = _mynorm(x3) - _mynorm(x1_ref[...])
    cat = jnp.concatenate([x0_ref[...], x1_ref[...], xx2_ref[...], xx3],
                          axis=1)
    out_ref[...] = _dot_t(cat, w4_ref[...]) + b4_ref[...]


def _node_spec(width):
    return pl.BlockSpec((BN, width), lambda i: (i, 0))


def _full_spec(shape):
    nd = len(shape)
    return pl.BlockSpec(shape, lambda i, _nd=nd: (0,) * _nd)


def _part_spec(width):
    return pl.BlockSpec((2, BN, width), lambda i: (0, i, 0))


_GRID = N // BN


def _nodes_out(widths):
    return [jax.ShapeDtypeStruct((N, w), jnp.float32) for w in widths]


_t1 = pl.pallas_call(
    _t1_body,
    grid=(_GRID,),
    in_specs=[_node_spec(D), _full_spec((H, D)), _full_spec((1, H)),
              _part_spec(8)],
    out_specs=[_node_spec(H), _node_spec(H), _node_spec(8)],
    out_shape=_nodes_out([H, H, 8]),
)

_t2 = pl.pallas_call(
    _t2_body,
    grid=(_GRID,),
    in_specs=[_part_spec(H), _node_spec(H), _node_spec(8),
              _full_spec((H, H)), _full_spec((1, H))],
    out_specs=[_node_spec(H), _node_spec(H)],
    out_shape=_nodes_out([H, H]),
)

_t3 = pl.pallas_call(
    _t3_body,
    grid=(_GRID,),
    in_specs=[_part_spec(H), _node_spec(H), _node_spec(8),
              _full_spec((H, H)), _full_spec((1, H)), _node_spec(H)],
    out_specs=[_node_spec(H), _node_spec(H)],
    out_shape=_nodes_out([H, H]),
)

_t4 = pl.pallas_call(
    _t4_body,
    grid=(_GRID,),
    in_specs=[_part_spec(H), _node_spec(H), _node_spec(8),
              _full_spec((H, H)), _full_spec((1, H)), _node_spec(H),
              _node_spec(H), _node_spec(H), _full_spec((O, 4 * H)),
              _full_spec((1, O))],
    out_specs=_node_spec(O),
    out_shape=jax.ShapeDtypeStruct((N, O), jnp.float32),
)


def kernel(x, edge_index, W1, b1, Wc1, bc1, Wc2, bc2, Wc3, bc3, W4, b4):
    pad = EP - E
    pad_i = jnp.arange(pad, dtype=edge_index.dtype)
    srcp = jnp.concatenate([edge_index[0], pad_i % N])
    dstp = jnp.concatenate([edge_index[1], N + pad_i % (R - N)])
    srcA = srcp[:EP_A].reshape(_NSUB, CH_A, K)
    srcB = srcp[EP_A:].reshape(_NSUB, CH_B, K)
    dstA = dstp[:EP_A].reshape(_NSUB, CH_A, K)
    dstB = dstp[EP_A:].reshape(_NSUB, CH_B, K)
    z32 = jnp.zeros((R, H), jnp.float32)
    z8 = jnp.zeros((R, 8), jnp.float32)
    ones8 = jnp.ones((K, 8), jnp.float32)

    degp = _deg(dstA, dstB, ones8, z8)
    x0, y0, dinv = _t1(x, W1, b1.reshape(1, H), degp)
    sp1 = _propagate(y0, srcA, dstA, srcB, dstB, z32)
    x1, y1 = _t2(sp1, y0, dinv, Wc1, bc1.reshape(1, H))
    sp2 = _propagate(y1, srcA, dstA, srcB, dstB, z32)
    xx2, y2 = _t3(sp2, y1, dinv, Wc2, bc2.reshape(1, H), x0)
    sp3 = _propagate(y2, srcA, dstA, srcB, dstB, z32)
    x4 = _t4(sp3, y2, dinv, Wc3, bc3.reshape(1, H), x0, x1, xx2,
             W4, b4.reshape(1, O))
    return x4
